# baseline (device time: 482216 ns/iter reference)
import functools

import jax
import jax.numpy as jnp
from jax import lax
from jax.experimental import pallas as pl
from jax.experimental.pallas import tpu as pltpu

N_DEV = 32
M = 2048
CHUNK = M // N_DEV


def kernel(x, w_mat):
    m, k_per = x.shape
    _, n = w_mat.shape

    def body(x_ref, w_ref, out_ref, init_ref, rbuf, ss_rs, rs_rs, ss_ag, rs_ag):
        my = lax.axis_index("i")
        right = lax.rem(my + 1, N_DEV)
        left = lax.rem(my + N_DEV - 1, N_DEV)

        barrier_sem = pltpu.get_barrier_semaphore()
        for nbr in (left, right):
            pl.semaphore_signal(
                barrier_sem, inc=1,
                device_id=(nbr,), device_id_type=pl.DeviceIdType.MESH,
            )
        pl.semaphore_wait(barrier_sem, 2)

        def partial_chunk(c):
            xs = x_ref[pl.ds(c * CHUNK, CHUNK), :]
            return jnp.dot(xs, w_ref[:, :], preferred_element_type=jnp.float32)

        init_ref[:, :] = partial_chunk(my)
        for s in range(N_DEV - 1):
            src = init_ref if s == 0 else rbuf.at[s - 1]
            rdma = pltpu.make_async_remote_copy(
                src_ref=src,
                dst_ref=rbuf.at[s],
                send_sem=ss_rs.at[s],
                recv_sem=rs_rs.at[s],
                device_id=(right,),
                device_id_type=pl.DeviceIdType.MESH,
            )
            rdma.start()
            rdma.wait()
            c = jnp.mod(my - s - 1, N_DEV)
            rbuf[s, :, :] = rbuf[s, :, :] + partial_chunk(c)

        own = jnp.mod(my + 1, N_DEV)
        red = rbuf[N_DEV - 2, :, :]
        out_ref[pl.ds(own * CHUNK, CHUNK), :] = red * jax.nn.sigmoid(red)

        for s in range(N_DEV - 1):
            c = jnp.mod(my + 1 - s, N_DEV)
            sl = (pl.ds(c * CHUNK, CHUNK), slice(None))
            rdma = pltpu.make_async_remote_copy(
                src_ref=out_ref.at[sl],
                dst_ref=out_ref.at[sl],
                send_sem=ss_ag.at[s],
                recv_sem=rs_ag.at[s],
                device_id=(right,),
                device_id_type=pl.DeviceIdType.MESH,
            )
            rdma.start()
            rdma.wait()

    return pl.pallas_call(
        body,
        out_shape=jax.ShapeDtypeStruct((M, n), jnp.float32),
        in_specs=[
            pl.BlockSpec(memory_space=pltpu.VMEM),
            pl.BlockSpec(memory_space=pltpu.VMEM),
        ],
        out_specs=pl.BlockSpec(memory_space=pltpu.VMEM),
        scratch_shapes=[
            pltpu.VMEM((CHUNK, n), jnp.float32),
            pltpu.VMEM((N_DEV - 1, CHUNK, n), jnp.float32),
            pltpu.SemaphoreType.DMA((N_DEV - 1,)),
            pltpu.SemaphoreType.DMA((N_DEV - 1,)),
            pltpu.SemaphoreType.DMA((N_DEV - 1,)),
            pltpu.SemaphoreType.DMA((N_DEV - 1,)),
        ],
        compiler_params=pltpu.CompilerParams(collective_id=0),
    )(x, w_mat)


# device time: 295805 ns/iter; 1.6302x vs baseline; 1.6302x over previous
import jax
import jax.numpy as jnp
from jax import lax
from jax.experimental import pallas as pl
from jax.experimental.pallas import tpu as pltpu

N_DEV = 32
M = 2048
CHUNK = M // N_DEV
HN = 1024


def _coords_of_rp(rp):
    q = jnp.where(rp < 16, rp, 31 - rp)
    x = jnp.where(rp < 16, 0, 1)
    z = q // 4
    r = q % 4
    y = jnp.where(z % 2 == 0, r, 3 - r)
    return x, y, z


def _mesh_of_coords(x, y, z):
    return z * 8 + y * 2 + jnp.where(y % 2 == 0, x, 1 - x)


def _rp_of_mesh(i):
    z = i // 8
    j = i % 8
    y = j // 2
    xb = j % 2
    x = jnp.where(y % 2 == 0, xb, 1 - xb)
    q = z * 4 + jnp.where(z % 2 == 0, y, 3 - y)
    return jnp.where(x == 0, q, 31 - q)


def kernel(x, w_mat):
    m, k_per = x.shape
    _, n = w_mat.shape

    def body(x_ref, w_ref, out_ref,
             initR, initL, rbufR, rbufL,
             ssR, rsR, ssL, rsL,
             agssR, agrsR, agssL, agrsL):
        my_mesh = lax.axis_index("i")
        rp = _rp_of_mesh(my_mesh)
        right = _mesh_of_coords(*_coords_of_rp(jnp.mod(rp + 1, N_DEV)))
        left = _mesh_of_coords(*_coords_of_rp(jnp.mod(rp + N_DEV - 1, N_DEV)))

        barrier_sem = pltpu.get_barrier_semaphore()
        for nbr in (left, right):
            pl.semaphore_signal(
                barrier_sem, inc=1,
                device_id=(nbr,), device_id_type=pl.DeviceIdType.MESH,
            )
        pl.semaphore_wait(barrier_sem, 2)

        def partialA(c):
            xs = x_ref[pl.ds(c * CHUNK, CHUNK), :]
            return jnp.dot(xs, w_ref[:, :HN], preferred_element_type=jnp.float32)

        def partialB(c):
            xs = x_ref[pl.ds(c * CHUNK, CHUNK), :]
            return jnp.dot(xs, w_ref[:, HN:], preferred_element_type=jnp.float32)

        drain = []

        initR[:, :] = partialA(rp)
        initL[:, :] = partialB(rp)
        for s in range(N_DEV - 1):
            sendR = pltpu.make_async_remote_copy(
                src_ref=initR if s == 0 else rbufR.at[s - 1],
                dst_ref=rbufR.at[s],
                send_sem=ssR.at[s], recv_sem=rsR.at[s],
                device_id=(right,), device_id_type=pl.DeviceIdType.MESH,
            )
            sendL = pltpu.make_async_remote_copy(
                src_ref=initL if s == 0 else rbufL.at[s - 1],
                dst_ref=rbufL.at[s],
                send_sem=ssL.at[s], recv_sem=rsL.at[s],
                device_id=(left,), device_id_type=pl.DeviceIdType.MESH,
            )
            sendR.start()
            sendL.start()
            drain.append(sendR)
            drain.append(sendL)
            cA = jnp.mod(rp - s - 1 + N_DEV, N_DEV)
            cB = jnp.mod(rp + s + 1, N_DEV)
            pA = partialA(cA)
            pB = partialB(cB)
            sendR.wait_recv()
            rbufR[s, :, :] = rbufR[s, :, :] + pA
            sendL.wait_recv()
            rbufL[s, :, :] = rbufL[s, :, :] + pB

        ownR = jnp.mod(rp + 1, N_DEV)
        ownL = jnp.mod(rp + N_DEV - 1, N_DEV)
        redR = rbufR[N_DEV - 2, :, :]
        redL = rbufL[N_DEV - 2, :, :]
        out_ref[pl.ds(ownR * CHUNK, CHUNK), :HN] = redR * jax.nn.sigmoid(redR)
        out_ref[pl.ds(ownL * CHUNK, CHUNK), HN:] = redL * jax.nn.sigmoid(redL)

        agR = []
        agL = []
        for s in range(N_DEV - 1):
            if s > 0:
                agR[s - 1].wait_recv()
                agL[s - 1].wait_recv()
            cA = jnp.mod(rp + 1 - s + N_DEV, N_DEV)
            cB = jnp.mod(rp - 1 + s + N_DEV, N_DEV)
            slA = (pl.ds(cA * CHUNK, CHUNK), pl.ds(0, HN))
            slB = (pl.ds(cB * CHUNK, CHUNK), pl.ds(HN, HN))
            gR = pltpu.make_async_remote_copy(
                src_ref=out_ref.at[slA], dst_ref=out_ref.at[slA],
                send_sem=agssR.at[s], recv_sem=agrsR.at[s],
                device_id=(right,), device_id_type=pl.DeviceIdType.MESH,
            )
            gL = pltpu.make_async_remote_copy(
                src_ref=out_ref.at[slB], dst_ref=out_ref.at[slB],
                send_sem=agssL.at[s], recv_sem=agrsL.at[s],
                device_id=(left,), device_id_type=pl.DeviceIdType.MESH,
            )
            gR.start()
            gL.start()
            agR.append(gR)
            agL.append(gL)
        agR[N_DEV - 2].wait_recv()
        agL[N_DEV - 2].wait_recv()

        for r in drain + agR + agL:
            r.wait_send()

    nsem = N_DEV - 1
    return pl.pallas_call(
        body,
        out_shape=jax.ShapeDtypeStruct((M, n), jnp.float32),
        in_specs=[
            pl.BlockSpec(memory_space=pltpu.VMEM),
            pl.BlockSpec(memory_space=pltpu.VMEM),
        ],
        out_specs=pl.BlockSpec(memory_space=pltpu.VMEM),
        scratch_shapes=[
            pltpu.VMEM((CHUNK, HN), jnp.float32),
            pltpu.VMEM((CHUNK, HN), jnp.float32),
            pltpu.VMEM((nsem, CHUNK, HN), jnp.float32),
            pltpu.VMEM((nsem, CHUNK, HN), jnp.float32),
            pltpu.SemaphoreType.DMA((nsem,)),
            pltpu.SemaphoreType.DMA((nsem,)),
            pltpu.SemaphoreType.DMA((nsem,)),
            pltpu.SemaphoreType.DMA((nsem,)),
            pltpu.SemaphoreType.DMA((nsem,)),
            pltpu.SemaphoreType.DMA((nsem,)),
            pltpu.SemaphoreType.DMA((nsem,)),
            pltpu.SemaphoreType.DMA((nsem,)),
        ],
        compiler_params=pltpu.CompilerParams(collective_id=0),
    )(x, w_mat)


# device time: 249025 ns/iter; 1.9364x vs baseline; 1.1879x over previous
import jax
import jax.numpy as jnp
from jax import lax
from jax.experimental import pallas as pl
from jax.experimental.pallas import tpu as pltpu

N_DEV = 32
N_PL = 16
M = 2048
CH = M // N_PL
HCH = CH // 2
HN = 1024


def _coords_of_mesh(i):
    z = i // 8
    j = i % 8
    y = j // 2
    xb = j % 2
    x = jnp.where(y % 2 == 0, xb, 1 - xb)
    return x, y, z


def _mesh_of_coords(x, y, z):
    return z * 8 + y * 2 + jnp.where(y % 2 == 0, x, 1 - x)


def _q_of_yz(y, z):
    q_lo = 3 * z + jnp.where(z % 2 == 0, y - 1, 3 - y)
    return jnp.where(y == 0, 15 - z, q_lo)


def _yz_of_q(p):
    z_lo = p // 3
    r = p % 3
    y_lo = jnp.where(z_lo % 2 == 0, 1 + r, 3 - r)
    y = jnp.where(p < 12, y_lo, 0)
    z = jnp.where(p < 12, z_lo, 15 - p)
    return y, z


def kernel(x, w_mat):
    m, k_per = x.shape
    _, n = w_mat.shape

    def body(x_ref, w_ref, out_ref,
             initR, initL, rbufR, rbufL, xbufR, xbufL,
             ssR, rsR, ssL, rsL,
             xssR, xrsR, xssL, xrsL,
             gssR, grsR, gssL, grsL,
             agssR, agrsR, agssL, agrsL):
        my_mesh = lax.axis_index("i")
        xc, yc, zc = _coords_of_mesh(my_mesh)
        q = _q_of_yz(yc, zc)

        yR, zR = _yz_of_q(jnp.mod(q + 1, N_PL))
        yL, zL = _yz_of_q(jnp.mod(q + N_PL - 1, N_PL))
        right = _mesh_of_coords(xc, yR, zR)
        left = _mesh_of_coords(xc, yL, zL)
        partner = _mesh_of_coords(1 - xc, yc, zc)

        barrier_sem = pltpu.get_barrier_semaphore()
        for nbr in (left, right, partner):
            pl.semaphore_signal(
                barrier_sem, inc=1,
                device_id=(nbr,), device_id_type=pl.DeviceIdType.MESH,
            )
        pl.semaphore_wait(barrier_sem, 3)

        def partialA(c):
            xs = x_ref[pl.ds(c * CH, CH), :]
            return jnp.dot(xs, w_ref[:, :HN], preferred_element_type=jnp.float32)

        def partialB(c):
            xs = x_ref[pl.ds(c * CH, CH), :]
            return jnp.dot(xs, w_ref[:, HN:], preferred_element_type=jnp.float32)

        drain = []

        initR[:, :] = partialA(q)
        initL[:, :] = partialB(q)
        for s in range(N_PL - 1):
            sendR = pltpu.make_async_remote_copy(
                src_ref=initR if s == 0 else rbufR.at[s - 1],
                dst_ref=rbufR.at[s],
                send_sem=ssR.at[s], recv_sem=rsR.at[s],
                device_id=(right,), device_id_type=pl.DeviceIdType.MESH,
            )
            sendL = pltpu.make_async_remote_copy(
                src_ref=initL if s == 0 else rbufL.at[s - 1],
                dst_ref=rbufL.at[s],
                send_sem=ssL.at[s], recv_sem=rsL.at[s],
                device_id=(left,), device_id_type=pl.DeviceIdType.MESH,
            )
            sendR.start()
            sendL.start()
            drain.append(sendR)
            drain.append(sendL)
            cA = jnp.mod(q - s - 1 + N_PL, N_PL)
            cB = jnp.mod(q + s + 1, N_PL)
            pA = partialA(cA)
            pB = partialB(cB)
            sendR.wait_recv()
            rbufR[s, :, :] = rbufR[s, :, :] + pA
            sendL.wait_recv()
            rbufL[s, :, :] = rbufL[s, :, :] + pB

        ownR = jnp.mod(q + 1, N_PL)
        ownL = jnp.mod(q + N_PL - 1, N_PL)
        keep = xc * HCH
        give = (1 - xc) * HCH
        last = N_PL - 2
        xR = pltpu.make_async_remote_copy(
            src_ref=rbufR.at[last, pl.ds(give, HCH), :],
            dst_ref=xbufR,
            send_sem=xssR, recv_sem=xrsR,
            device_id=(partner,), device_id_type=pl.DeviceIdType.MESH,
        )
        xL = pltpu.make_async_remote_copy(
            src_ref=rbufL.at[last, pl.ds(give, HCH), :],
            dst_ref=xbufL,
            send_sem=xssL, recv_sem=xrsL,
            device_id=(partner,), device_id_type=pl.DeviceIdType.MESH,
        )
        xR.start()
        xL.start()
        drain.append(xR)
        drain.append(xL)

        rowsR = ownR * CH + keep
        rowsL = ownL * CH + keep
        xR.wait_recv()
        redR = rbufR[last, pl.ds(keep, HCH), :] + xbufR[:, :]
        out_ref[pl.ds(rowsR, HCH), :HN] = redR * jax.nn.sigmoid(redR)
        xL.wait_recv()
        redL = rbufL[last, pl.ds(keep, HCH), :] + xbufL[:, :]
        out_ref[pl.ds(rowsL, HCH), HN:] = redL * jax.nn.sigmoid(redL)

        gR = pltpu.make_async_remote_copy(
            src_ref=out_ref.at[pl.ds(rowsR, HCH), pl.ds(0, HN)],
            dst_ref=out_ref.at[pl.ds(rowsR, HCH), pl.ds(0, HN)],
            send_sem=gssR, recv_sem=grsR,
            device_id=(partner,), device_id_type=pl.DeviceIdType.MESH,
        )
        gL = pltpu.make_async_remote_copy(
            src_ref=out_ref.at[pl.ds(rowsL, HCH), pl.ds(HN, HN)],
            dst_ref=out_ref.at[pl.ds(rowsL, HCH), pl.ds(HN, HN)],
            send_sem=gssL, recv_sem=grsL,
            device_id=(partner,), device_id_type=pl.DeviceIdType.MESH,
        )
        gR.start()
        gL.start()
        drain.append(gR)
        drain.append(gL)
        gR.wait_recv()
        gL.wait_recv()

        agR = []
        agL = []
        for s in range(N_PL - 1):
            if s > 0:
                agR[s - 1].wait_recv()
                agL[s - 1].wait_recv()
            cA = jnp.mod(q + 1 - s + N_PL, N_PL)
            cB = jnp.mod(q - 1 + s + N_PL, N_PL)
            slA = (pl.ds(cA * CH, CH), pl.ds(0, HN))
            slB = (pl.ds(cB * CH, CH), pl.ds(HN, HN))
            aR = pltpu.make_async_remote_copy(
                src_ref=out_ref.at[slA], dst_ref=out_ref.at[slA],
                send_sem=agssR.at[s], recv_sem=agrsR.at[s],
                device_id=(right,), device_id_type=pl.DeviceIdType.MESH,
            )
            aL = pltpu.make_async_remote_copy(
                src_ref=out_ref.at[slB], dst_ref=out_ref.at[slB],
                send_sem=agssL.at[s], recv_sem=agrsL.at[s],
                device_id=(left,), device_id_type=pl.DeviceIdType.MESH,
            )
            aR.start()
            aL.start()
            agR.append(aR)
            agL.append(aL)
        agR[N_PL - 2].wait_recv()
        agL[N_PL - 2].wait_recv()

        for r in drain + agR + agL:
            r.wait_send()

    nsem = N_PL - 1
    return pl.pallas_call(
        body,
        out_shape=jax.ShapeDtypeStruct((M, n), jnp.float32),
        in_specs=[
            pl.BlockSpec(memory_space=pltpu.VMEM),
            pl.BlockSpec(memory_space=pltpu.VMEM),
        ],
        out_specs=pl.BlockSpec(memory_space=pltpu.VMEM),
        scratch_shapes=[
            pltpu.VMEM((CH, HN), jnp.float32),
            pltpu.VMEM((CH, HN), jnp.float32),
            pltpu.VMEM((nsem, CH, HN), jnp.float32),
            pltpu.VMEM((nsem, CH, HN), jnp.float32),
            pltpu.VMEM((HCH, HN), jnp.float32),
            pltpu.VMEM((HCH, HN), jnp.float32),
            pltpu.SemaphoreType.DMA((nsem,)),
            pltpu.SemaphoreType.DMA((nsem,)),
            pltpu.SemaphoreType.DMA((nsem,)),
            pltpu.SemaphoreType.DMA((nsem,)),
            pltpu.SemaphoreType.DMA,
            pltpu.SemaphoreType.DMA,
            pltpu.SemaphoreType.DMA,
            pltpu.SemaphoreType.DMA,
            pltpu.SemaphoreType.DMA,
            pltpu.SemaphoreType.DMA,
            pltpu.SemaphoreType.DMA,
            pltpu.SemaphoreType.DMA,
            pltpu.SemaphoreType.DMA((nsem,)),
            pltpu.SemaphoreType.DMA((nsem,)),
            pltpu.SemaphoreType.DMA((nsem,)),
            pltpu.SemaphoreType.DMA((nsem,)),
        ],
        compiler_params=pltpu.CompilerParams(collective_id=0),
    )(x, w_mat)


# device time: 200048 ns/iter; 2.4105x vs baseline; 1.2448x over previous
import jax
import jax.numpy as jnp
from jax import lax
from jax.experimental import pallas as pl
from jax.experimental.pallas import tpu as pltpu

N_DEV = 32
N_PL = 16
M = 2048
CH = M // N_PL
HCH = CH // 2
HN = 1024
P = 2
PCH = CH // P
XP = 2
XPCH = HCH // XP


def _coords_of_mesh(i):
    z = i // 8
    j = i % 8
    y = j // 2
    xb = j % 2
    x = jnp.where(y % 2 == 0, xb, 1 - xb)
    return x, y, z


def _mesh_of_coords(x, y, z):
    return z * 8 + y * 2 + jnp.where(y % 2 == 0, x, 1 - x)


def _q_of_yz(y, z):
    q_lo = 3 * z + jnp.where(z % 2 == 0, y - 1, 3 - y)
    return jnp.where(y == 0, 15 - z, q_lo)


def _yz_of_q(p):
    z_lo = p // 3
    r = p % 3
    y_lo = jnp.where(z_lo % 2 == 0, 1 + r, 3 - r)
    y = jnp.where(p < 12, y_lo, 0)
    z = jnp.where(p < 12, z_lo, 15 - p)
    return y, z


def kernel(x, w_mat):
    m, k_per = x.shape
    _, n = w_mat.shape

    def body(x_ref, w_ref, out_ref,
             initR, initL, rbufR, rbufL, xbufR, xbufL,
             ssR, rsR, ssL, rsL,
             xssR, xrsR, xssL, xrsL,
             gssR, grsR, gssL, grsL,
             agssR, agrsR, agssL, agrsL):
        my_mesh = lax.axis_index("i")
        xc, yc, zc = _coords_of_mesh(my_mesh)
        q = _q_of_yz(yc, zc)

        yR, zR = _yz_of_q(jnp.mod(q + 1, N_PL))
        yL, zL = _yz_of_q(jnp.mod(q + N_PL - 1, N_PL))
        right = _mesh_of_coords(xc, yR, zR)
        left = _mesh_of_coords(xc, yL, zL)
        partner = _mesh_of_coords(1 - xc, yc, zc)

        barrier_sem = pltpu.get_barrier_semaphore()
        for nbr in (left, right, partner):
            pl.semaphore_signal(
                barrier_sem, inc=1,
                device_id=(nbr,), device_id_type=pl.DeviceIdType.MESH,
            )
        pl.semaphore_wait(barrier_sem, 3)

        def partialA(c):
            xs = x_ref[pl.ds(c * CH, CH), :]
            return jnp.dot(xs, w_ref[:, :HN], preferred_element_type=jnp.float32)

        def partialB(c):
            xs = x_ref[pl.ds(c * CH, CH), :]
            return jnp.dot(xs, w_ref[:, HN:], preferred_element_type=jnp.float32)

        drain = []

        initR[:, :] = partialA(q)
        initL[:, :] = partialB(q)
        rsR_d = []
        rsL_d = []
        pA_prev = None
        pB_prev = None
        for s in range(N_PL - 1):
            curR = []
            curL = []
            for p in range(P):
                rows = pl.ds(p * PCH, PCH)
                if s > 0:
                    rsR_d[s - 1][p].wait_recv()
                    rbufR[s - 1, rows, :] = (
                        rbufR[s - 1, rows, :] + pA_prev[p * PCH:(p + 1) * PCH, :]
                    )
                srcR = initR.at[rows, :] if s == 0 else rbufR.at[s - 1, rows, :]
                dR = pltpu.make_async_remote_copy(
                    src_ref=srcR,
                    dst_ref=rbufR.at[s, rows, :],
                    send_sem=ssR.at[s * P + p], recv_sem=rsR.at[s * P + p],
                    device_id=(right,), device_id_type=pl.DeviceIdType.MESH,
                )
                dR.start()
                curR.append(dR)
                if s > 0:
                    rsL_d[s - 1][p].wait_recv()
                    rbufL[s - 1, rows, :] = (
                        rbufL[s - 1, rows, :] + pB_prev[p * PCH:(p + 1) * PCH, :]
                    )
                srcL = initL.at[rows, :] if s == 0 else rbufL.at[s - 1, rows, :]
                dL = pltpu.make_async_remote_copy(
                    src_ref=srcL,
                    dst_ref=rbufL.at[s, rows, :],
                    send_sem=ssL.at[s * P + p], recv_sem=rsL.at[s * P + p],
                    device_id=(left,), device_id_type=pl.DeviceIdType.MESH,
                )
                dL.start()
                curL.append(dL)
            rsR_d.append(curR)
            rsL_d.append(curL)
            drain.extend(curR)
            drain.extend(curL)
            pA_prev = partialA(jnp.mod(q - s - 1 + N_PL, N_PL))
            pB_prev = partialB(jnp.mod(q + s + 1, N_PL))

        last = N_PL - 2
        for p in range(P):
            rows = pl.ds(p * PCH, PCH)
            rsR_d[last][p].wait_recv()
            rbufR[last, rows, :] = (
                rbufR[last, rows, :] + pA_prev[p * PCH:(p + 1) * PCH, :]
            )
            rsL_d[last][p].wait_recv()
            rbufL[last, rows, :] = (
                rbufL[last, rows, :] + pB_prev[p * PCH:(p + 1) * PCH, :]
            )

        ownR = jnp.mod(q + 1, N_PL)
        ownL = jnp.mod(q + N_PL - 1, N_PL)
        keep = xc * HCH
        give = (1 - xc) * HCH
        xR_d = []
        xL_d = []
        for p in range(XP):
            dR = pltpu.make_async_remote_copy(
                src_ref=rbufR.at[last, pl.ds(give + p * XPCH, XPCH), :],
                dst_ref=xbufR.at[pl.ds(p * XPCH, XPCH), :],
                send_sem=xssR.at[p], recv_sem=xrsR.at[p],
                device_id=(partner,), device_id_type=pl.DeviceIdType.MESH,
            )
            dL = pltpu.make_async_remote_copy(
                src_ref=rbufL.at[last, pl.ds(give + p * XPCH, XPCH), :],
                dst_ref=xbufL.at[pl.ds(p * XPCH, XPCH), :],
                send_sem=xssL.at[p], recv_sem=xrsL.at[p],
                device_id=(partner,), device_id_type=pl.DeviceIdType.MESH,
            )
            dR.start()
            dL.start()
            xR_d.append(dR)
            xL_d.append(dL)
        drain.extend(xR_d)
        drain.extend(xL_d)

        rowsR = ownR * CH + keep
        rowsL = ownL * CH + keep
        gR_d = []
        gL_d = []
        for p in range(XP):
            prow = pl.ds(p * XPCH, XPCH)
            xR_d[p].wait_recv()
            redR = rbufR[last, pl.ds(keep + p * XPCH, XPCH), :] + xbufR[prow, :]
            out_ref[pl.ds(rowsR + p * XPCH, XPCH), :HN] = (
                redR * jax.nn.sigmoid(redR)
            )
            dgR = pltpu.make_async_remote_copy(
                src_ref=out_ref.at[pl.ds(rowsR + p * XPCH, XPCH), pl.ds(0, HN)],
                dst_ref=out_ref.at[pl.ds(rowsR + p * XPCH, XPCH), pl.ds(0, HN)],
                send_sem=gssR.at[p], recv_sem=grsR.at[p],
                device_id=(partner,), device_id_type=pl.DeviceIdType.MESH,
            )
            dgR.start()
            gR_d.append(dgR)

            xL_d[p].wait_recv()
            redL = rbufL[last, pl.ds(keep + p * XPCH, XPCH), :] + xbufL[prow, :]
            out_ref[pl.ds(rowsL + p * XPCH, XPCH), HN:] = (
                redL * jax.nn.sigmoid(redL)
            )
            dgL = pltpu.make_async_remote_copy(
                src_ref=out_ref.at[pl.ds(rowsL + p * XPCH, XPCH), pl.ds(HN, HN)],
                dst_ref=out_ref.at[pl.ds(rowsL + p * XPCH, XPCH), pl.ds(HN, HN)],
                send_sem=gssL.at[p], recv_sem=grsL.at[p],
                device_id=(partner,), device_id_type=pl.DeviceIdType.MESH,
            )
            dgL.start()
            gL_d.append(dgL)
        drain.extend(gR_d)
        drain.extend(gL_d)
        for p in range(XP):
            gR_d[p].wait_recv()
            gL_d[p].wait_recv()

        agR_d = []
        agL_d = []
        for s in range(N_PL - 1):
            cA = jnp.mod(q + 1 - s + N_PL, N_PL)
            cB = jnp.mod(q - 1 + s + N_PL, N_PL)
            curR = []
            curL = []
            for p in range(P):
                if s > 0:
                    agR_d[s - 1][p].wait_recv()
                    agL_d[s - 1][p].wait_recv()
                slA = (pl.ds(cA * CH + p * PCH, PCH), pl.ds(0, HN))
                slB = (pl.ds(cB * CH + p * PCH, PCH), pl.ds(HN, HN))
                aR = pltpu.make_async_remote_copy(
                    src_ref=out_ref.at[slA], dst_ref=out_ref.at[slA],
                    send_sem=agssR.at[s * P + p], recv_sem=agrsR.at[s * P + p],
                    device_id=(right,), device_id_type=pl.DeviceIdType.MESH,
                )
                aL = pltpu.make_async_remote_copy(
                    src_ref=out_ref.at[slB], dst_ref=out_ref.at[slB],
                    send_sem=agssL.at[s * P + p], recv_sem=agrsL.at[s * P + p],
                    device_id=(left,), device_id_type=pl.DeviceIdType.MESH,
                )
                aR.start()
                aL.start()
                curR.append(aR)
                curL.append(aL)
            agR_d.append(curR)
            agL_d.append(curL)
            drain.extend(curR)
            drain.extend(curL)
        for p in range(P):
            agR_d[N_PL - 2][p].wait_recv()
            agL_d[N_PL - 2][p].wait_recv()

        for r in drain:
            r.wait_send()

    nsem = (N_PL - 1) * P
    return pl.pallas_call(
        body,
        out_shape=jax.ShapeDtypeStruct((M, n), jnp.float32),
        in_specs=[
            pl.BlockSpec(memory_space=pltpu.VMEM),
            pl.BlockSpec(memory_space=pltpu.VMEM),
        ],
        out_specs=pl.BlockSpec(memory_space=pltpu.VMEM),
        scratch_shapes=[
            pltpu.VMEM((CH, HN), jnp.float32),
            pltpu.VMEM((CH, HN), jnp.float32),
            pltpu.VMEM((N_PL - 1, CH, HN), jnp.float32),
            pltpu.VMEM((N_PL - 1, CH, HN), jnp.float32),
            pltpu.VMEM((HCH, HN), jnp.float32),
            pltpu.VMEM((HCH, HN), jnp.float32),
            pltpu.SemaphoreType.DMA((nsem,)),
            pltpu.SemaphoreType.DMA((nsem,)),
            pltpu.SemaphoreType.DMA((nsem,)),
            pltpu.SemaphoreType.DMA((nsem,)),
            pltpu.SemaphoreType.DMA((XP,)),
            pltpu.SemaphoreType.DMA((XP,)),
            pltpu.SemaphoreType.DMA((XP,)),
            pltpu.SemaphoreType.DMA((XP,)),
            pltpu.SemaphoreType.DMA((XP,)),
            pltpu.SemaphoreType.DMA((XP,)),
            pltpu.SemaphoreType.DMA((XP,)),
            pltpu.SemaphoreType.DMA((XP,)),
            pltpu.SemaphoreType.DMA((nsem,)),
            pltpu.SemaphoreType.DMA((nsem,)),
            pltpu.SemaphoreType.DMA((nsem,)),
            pltpu.SemaphoreType.DMA((nsem,)),
        ],
        compiler_params=pltpu.CompilerParams(collective_id=0),
    )(x, w_mat)
